# Initial kernel scaffold; baseline (speedup 1.0000x reference)
import functools

import jax
import jax.numpy as jnp
from jax import lax
from jax.experimental import pallas as pl
from jax.experimental.pallas import tpu as pltpu
from jax.experimental.pallas import tpu_sc as plsc

PITCH_NUM = 128
BEAT_NUM = 64
EMB = 128

NC = 2
NS = 16
L = 16
NW = NC * NS

BLK = 128
ROWS_BLK = 3 * BLK


@functools.cache
def _build(ntok: int):
    assert ntok % (NW * BLK) == 0
    tpw = ntok // NW
    nblk = tpw // BLK

    mesh = plsc.VectorSubcoreMesh(core_axis_name="c", subcore_axis_name="s")

    @functools.partial(
        pl.kernel,
        out_type=jax.ShapeDtypeStruct((3 * ntok, EMB), jnp.float32),
        mesh=mesh,
        scratch_types=[
            pltpu.VMEM((4 * BLK,), jnp.int32),
            pltpu.VMEM((3, BLK), jnp.int32),
            pltpu.VMEM((ROWS_BLK, EMB), jnp.float32),
            pltpu.SemaphoreType.DMA,
        ],
    )
    def emb_kernel(tbl_hbm, x_hbm, out_hbm, x_v, idx_v, rows_v, sem):
        wid = lax.axis_index("s") * NC + lax.axis_index("c")
        lane = lax.broadcasted_iota(jnp.int32, (L,), 0)

        def block(j, carry):
            tok0 = wid * tpw + j * BLK
            pltpu.sync_copy(x_hbm.at[pl.ds(tok0 * 4, 4 * BLK)], x_v)
            for g in range(BLK // L):
                t = g * L + lane
                for f, off in ((1, 0), (2, PITCH_NUM), (3, PITCH_NUM + BEAT_NUM)):
                    vals = plsc.load_gather(x_v, [t * 4 + f]) + off
                    pos = t * 3 + (f - 1)
                    plsc.store_scatter(idx_v, [pos >> 7, pos & 127], vals)
            copies = [
                pltpu.async_copy(
                    tbl_hbm.at[idx_v.at[k]],
                    rows_v.at[pl.ds(k * BLK, BLK)],
                    sem,
                )
                for k in range(3)
            ]
            for cp in copies:
                cp.wait()
            pltpu.sync_copy(rows_v, out_hbm.at[pl.ds(tok0 * 3, ROWS_BLK)])
            return carry

        lax.fori_loop(0, nblk, block, 0)

    return emb_kernel


def kernel(x, pitch_embedding, beat_embedding, dur_embedding):
    b, s, _ = x.shape
    ntok = b * s
    tbl = jnp.concatenate(
        [pitch_embedding, beat_embedding, dur_embedding], axis=0
    ).astype(jnp.float32)
    x_flat = x.astype(jnp.int32).reshape(-1)
    out = _build(ntok)(tbl, x_flat)
    return out.reshape(b, s, 3 * EMB)


# speedup vs baseline: 2.1845x; 2.1845x over previous
import functools

import jax
import jax.numpy as jnp
from jax import lax
from jax.experimental import pallas as pl
from jax.experimental.pallas import tpu as pltpu
from jax.experimental.pallas import tpu_sc as plsc

PITCH_NUM = 128
BEAT_NUM = 64
EMB = 128

NC = 2
NS = 16
L = 16
NW = NC * NS

BLK = 128
ROWS_BLK = 3 * BLK


@functools.cache
def _build(ntok: int):
    assert ntok % (NW * BLK) == 0
    tpw = ntok // NW
    nblk = tpw // BLK

    mesh = plsc.VectorSubcoreMesh(core_axis_name="c", subcore_axis_name="s")

    @functools.partial(
        pl.kernel,
        out_type=jax.ShapeDtypeStruct((3 * ntok, EMB), jnp.float32),
        mesh=mesh,
        compiler_params=pltpu.CompilerParams(needs_layout_passes=False),
        scratch_types=[
            pltpu.VMEM((4 * BLK,), jnp.int32),
            pltpu.VMEM((3, BLK), jnp.int32),
            pltpu.VMEM((ROWS_BLK, EMB), jnp.float32),
            pltpu.SemaphoreType.DMA,
        ],
    )
    def emb_kernel(tbl_hbm, x_hbm, out_hbm, x_v, idx_v, rows_v, sem):
        wid = lax.axis_index("s") * NC + lax.axis_index("c")
        lane = lax.broadcasted_iota(jnp.int32, (L,), 0)

        def block(j, carry):
            tok0 = wid * tpw + j * BLK
            pltpu.sync_copy(x_hbm.at[pl.ds(tok0 * 4, 4 * BLK)], x_v)
            for g in range(BLK // L):
                t = g * L + lane
                for f, off in ((1, 0), (2, PITCH_NUM), (3, PITCH_NUM + BEAT_NUM)):
                    vals = plsc.load_gather(x_v, [t * 4 + f]) + off
                    pos = t * 3 + (f - 1)
                    plsc.store_scatter(idx_v, [pos >> 7, pos & 127], vals)
            copies = [
                pltpu.async_copy(
                    tbl_hbm.at[idx_v.at[k]],
                    rows_v.at[pl.ds(k * BLK, BLK)],
                    sem,
                )
                for k in range(3)
            ]
            for cp in copies:
                cp.wait()
            pltpu.sync_copy(rows_v, out_hbm.at[pl.ds(tok0 * 3, ROWS_BLK)])
            return carry

        lax.fori_loop(0, nblk, block, 0)

    return emb_kernel


def kernel(x, pitch_embedding, beat_embedding, dur_embedding):
    b, s, _ = x.shape
    ntok = b * s
    tbl = jnp.concatenate(
        [pitch_embedding, beat_embedding, dur_embedding], axis=0
    ).astype(jnp.float32)
    x_flat = x.astype(jnp.int32).reshape(-1)
    out = _build(ntok)(tbl, x_flat)
    return out.reshape(b, s, 3 * EMB)
